# Initial kernel scaffold; baseline (speedup 1.0000x reference)
import functools

import jax
import jax.numpy as jnp
from jax import lax
from jax.experimental import pallas as pl
from jax.experimental.pallas import tpu as pltpu

_B = 1024
_EMB = 128
_K = 10
_RT = 256


def _dist_topk_body(t_ref, a_ref, v_ref, tt_ref, at_ref, vt_ref, idx_ref,
                    h_scr, ht_scr, acc_scr):
    step = pl.program_id(0)

    @pl.when(step == 0)
    def _():
        h_scr[...] = (t_ref[...] + a_ref[...] + v_ref[...]) * (1.0 / 3.0)
        ht_scr[...] = (tt_ref[...] + at_ref[...] + vt_ref[...]) * (1.0 / 3.0)

    x = h_scr[pl.ds(step * _RT, _RT), :]
    ht = ht_scr[...]

    acc_scr[...] = jnp.zeros((_RT, _B), jnp.float32)
    UK = 4

    def kbody(kk, _):
        k0 = kk * UK
        part = jnp.zeros((_RT, _B), jnp.float32)
        for u in range(UK):
            xk = lax.dynamic_slice(x, (0, k0 + u), (_RT, 1))
            yk = lax.dynamic_slice(ht, (k0 + u, 0), (1, _B))
            part = part + jnp.abs(xk - yk)
        acc_scr[...] = acc_scr[...] + part
        return 0

    lax.fori_loop(0, _EMB // UK, kbody, 0)

    lanes = lax.broadcasted_iota(jnp.int32, (_RT, _B), 1)
    idx_ref[...] = jnp.zeros((_RT, 16), jnp.int32)
    for j in range(_K):
        acc = acc_scr[...]
        m = jnp.max(acc, axis=1, keepdims=True)
        cand = jnp.where(acc >= m, lanes, _B)
        idx = jnp.min(cand, axis=1, keepdims=True)
        idx_ref[:, j:j + 1] = idx
        acc_scr[...] = jnp.where(lanes == idx, -1.0, acc)


def _adjacency_body(idx_ref, out_ref, ct_scr, a_scr):
    i = pl.program_id(0)
    j = pl.program_id(1)

    @pl.when((i == 0) & (j == 0))
    def _():
        ii = lax.broadcasted_iota(jnp.int32, (_B, _B), 0)
        jj = lax.broadcasted_iota(jnp.int32, (_B, _B), 1)
        cond = ii == jj
        for c in range(_K):
            cond = cond | (jj == idx_ref[:, c:c + 1])
        ct = jnp.where(cond, 1.0, 0.0)
        ct_scr[...] = ct
        s = jnp.sum(ct, axis=1, keepdims=True)
        w = ct / s
        a0 = lax.dot_general(w, ct, (((0,), (0,)), ((), ())),
                             preferred_element_type=jnp.float32)
        ones = jnp.ones((_B, 1), jnp.float32)
        dcol = lax.dot_general(ct, ones, (((0,), (0,)), ((), ())))
        eye = jnp.where(ii == jj, 1.0, 0.0)
        a_scr[...] = (a0 + eye) / (3.0 * (1.0 + dcol))

    out_ref[...] = a_scr[...]


def _build_calls():
    full_spec = lambda shape: pl.BlockSpec(shape, lambda s: (0, 0))
    dist_topk = pl.pallas_call(
        _dist_topk_body,
        grid=(_B // _RT,),
        in_specs=[
            full_spec((_B, _EMB)), full_spec((_B, _EMB)), full_spec((_B, _EMB)),
            full_spec((_EMB, _B)), full_spec((_EMB, _B)), full_spec((_EMB, _B)),
        ],
        out_specs=pl.BlockSpec((_RT, 16), lambda s: (s, 0)),
        out_shape=jax.ShapeDtypeStruct((_B, 16), jnp.int32),
        scratch_shapes=[
            pltpu.VMEM((_B, _EMB), jnp.float32),
            pltpu.VMEM((_EMB, _B), jnp.float32),
            pltpu.VMEM((_RT, _B), jnp.float32),
        ],
    )
    adjacency = pl.pallas_call(
        _adjacency_body,
        grid=(3, 3),
        in_specs=[pl.BlockSpec((_B, 16), lambda i, j: (0, 0))],
        out_specs=pl.BlockSpec((_B, _B), lambda i, j: (i, j)),
        out_shape=jax.ShapeDtypeStruct((3 * _B, 3 * _B), jnp.float32),
        scratch_shapes=[
            pltpu.VMEM((_B, _B), jnp.float32),
            pltpu.VMEM((_B, _B), jnp.float32),
        ],
    )
    return dist_topk, adjacency


_DIST_TOPK, _ADJACENCY = _build_calls()


@jax.jit
def kernel(nodes_t, nodes_a, nodes_v, batch_size):
    del batch_size
    t = nodes_t.astype(jnp.float32)
    a = nodes_a.astype(jnp.float32)
    v = nodes_v.astype(jnp.float32)
    idx = _DIST_TOPK(t, a, v, t.T, a.T, v.T)
    adjacency = _ADJACENCY(idx)
    nodes_list = jnp.concatenate([t, a, v], axis=0)
    return adjacency, nodes_list


# speedup vs baseline: 11.5760x; 11.5760x over previous
import functools

import jax
import jax.numpy as jnp
from jax import lax
from jax.experimental import pallas as pl
from jax.experimental.pallas import tpu as pltpu

_B = 1024
_EMB = 128
_K = 10
_RT = 256


def _dist_topk_body(t_ref, a_ref, v_ref, tt_ref, at_ref, vt_ref, idx_ref,
                    h_scr, ht_scr, acc_scr):
    step = pl.program_id(0)

    C = jnp.float32(0.333984375)

    def _h(x, y, z):
        xb = x.astype(jnp.bfloat16).astype(jnp.float32)
        yb = y.astype(jnp.bfloat16).astype(jnp.float32)
        zb = z.astype(jnp.bfloat16).astype(jnp.float32)
        return (C * xb + C * yb) + C * zb

    @pl.when(step == 0)
    def _():
        h_scr[...] = _h(t_ref[...], a_ref[...], v_ref[...])
        ht_scr[...] = _h(tt_ref[...], at_ref[...], vt_ref[...])

    x = h_scr[pl.ds(step * _RT, _RT), :]

    acc_scr[...] = jnp.zeros((_RT, _B), jnp.float32)
    UK = 8
    sub = lax.broadcasted_iota(jnp.int32, (UK, _EMB), 0)
    lane = lax.broadcasted_iota(jnp.int32, (UK, _EMB), 1)

    def kbody(kk, _):
        yblk = ht_scr[pl.ds(kk * UK, UK), :]
        oh = jnp.where(lane == kk * UK + sub, 1.0, 0.0)
        xblk = lax.dot_general(x, oh, (((1,), (1,)), ((), ())),
                               precision=lax.Precision.HIGHEST,
                               preferred_element_type=jnp.float32)
        part = jnp.zeros((_RT, _B), jnp.float32)
        for u in range(UK):
            xk = lax.slice(xblk, (0, u), (_RT, u + 1))
            yk = lax.slice(yblk, (u, 0), (u + 1, _B))
            part = part + jnp.abs(xk - yk)
        acc_scr[...] = acc_scr[...] + part
        return 0

    lax.fori_loop(0, _EMB // UK, kbody, 0)

    lanes = lax.broadcasted_iota(jnp.int32, (_RT, _B), 1)
    idx_ref[...] = jnp.zeros((_RT, 16), jnp.int32)
    for j in range(_K):
        acc = acc_scr[...]
        m = jnp.max(acc, axis=1, keepdims=True)
        cand = jnp.where(acc >= m, lanes, _B)
        idx = jnp.min(cand, axis=1, keepdims=True)
        idx_ref[:, j:j + 1] = idx
        acc_scr[...] = jnp.where(lanes == idx, -1.0, acc)


def _adjacency_body(idx_ref, out_ref, ct_scr, a_scr):
    i = pl.program_id(0)
    j = pl.program_id(1)

    @pl.when((i == 0) & (j == 0))
    def _():
        ii = lax.broadcasted_iota(jnp.int32, (_B, _B), 0)
        jj = lax.broadcasted_iota(jnp.int32, (_B, _B), 1)
        cond = ii == jj
        for c in range(_K):
            cond = cond | (jj == idx_ref[:, c:c + 1])
        ct = jnp.where(cond, 1.0, 0.0)
        ct_scr[...] = ct
        s = jnp.sum(ct, axis=1, keepdims=True)
        w = ct / s
        a0 = lax.dot_general(w, ct, (((0,), (0,)), ((), ())),
                             precision=lax.Precision.HIGHEST,
                             preferred_element_type=jnp.float32)
        ones = jnp.ones((_B, 1), jnp.float32)
        dcol = lax.dot_general(ct, ones, (((0,), (0,)), ((), ())),
                               precision=lax.Precision.HIGHEST)
        eye = jnp.where(ii == jj, 1.0, 0.0)
        a_scr[...] = (a0 + eye) / (3.0 * (1.0 + dcol))

    out_ref[...] = a_scr[...]


def _build_calls():
    full_spec = lambda shape: pl.BlockSpec(shape, lambda s: (0, 0))
    dist_topk = pl.pallas_call(
        _dist_topk_body,
        grid=(_B // _RT,),
        in_specs=[
            full_spec((_B, _EMB)), full_spec((_B, _EMB)), full_spec((_B, _EMB)),
            full_spec((_EMB, _B)), full_spec((_EMB, _B)), full_spec((_EMB, _B)),
        ],
        out_specs=pl.BlockSpec((_RT, 16), lambda s: (s, 0)),
        out_shape=jax.ShapeDtypeStruct((_B, 16), jnp.int32),
        scratch_shapes=[
            pltpu.VMEM((_B, _EMB), jnp.float32),
            pltpu.VMEM((_EMB, _B), jnp.float32),
            pltpu.VMEM((_RT, _B), jnp.float32),
        ],
    )
    adjacency = pl.pallas_call(
        _adjacency_body,
        grid=(3, 3),
        in_specs=[pl.BlockSpec((_B, 16), lambda i, j: (0, 0))],
        out_specs=pl.BlockSpec((_B, _B), lambda i, j: (i, j)),
        out_shape=jax.ShapeDtypeStruct((3 * _B, 3 * _B), jnp.float32),
        scratch_shapes=[
            pltpu.VMEM((_B, _B), jnp.float32),
            pltpu.VMEM((_B, _B), jnp.float32),
        ],
    )
    return dist_topk, adjacency


_DIST_TOPK, _ADJACENCY = _build_calls()


@jax.jit
def kernel(nodes_t, nodes_a, nodes_v, batch_size):
    del batch_size
    t = nodes_t.astype(jnp.float32)
    a = nodes_a.astype(jnp.float32)
    v = nodes_v.astype(jnp.float32)
    idx = _DIST_TOPK(t, a, v, t.T, a.T, v.T)
    adjacency = _ADJACENCY(idx)
    nodes_list = jnp.concatenate([t, a, v], axis=0)
    return adjacency, nodes_list
